# baseline (device time: 219797 ns/iter reference)
import jax
import jax.numpy as jnp
from jax import lax
from jax.experimental import pallas as pl
from jax.experimental.pallas import tpu as pltpu

N_DEV = 16
SQ = 2048
DM = 1024
CHUNK = SQ // N_DEV


HALF = DM // 2
QROWS = SQ // 4
SUB = SQ // 16


def _allreduce_body(
    ctx_ref, wo_ref, out_ref, stagA_r, stagA_l, stagB_r, stagB_l,
    sA_r, rA_r, sA_l, rA_l,
    sB_r, rB_r, sB_l, rB_l,
    sC_r, rC_r, sC_l, rC_l,
):
    my = lax.axis_index("i")
    zi = my // 4
    pi = lax.rem(my, 4)
    plane_r = zi * 4 + lax.rem(pi + 1, 4)
    plane_l = zi * 4 + lax.rem(pi + 3, 4)
    z_r = lax.rem(zi + 1, 4) * 4 + pi
    z_l = lax.rem(zi + 3, 4) * 4 + pi

    barrier = pltpu.get_barrier_semaphore()
    for nbr in (plane_l, plane_r, z_l, z_r):
        pl.semaphore_signal(
            barrier, inc=1, device_id=(nbr,), device_id_type=pl.DeviceIdType.MESH
        )
    pl.semaphore_wait(barrier, 4)

    for c in range(8):
        out_ref[pl.ds(c * 256, 256), :] = lax.dot_general(
            ctx_ref[pl.ds(c * 256, 256), :], wo_ref[...],
            (((1,), (0,)), ((), ())), preferred_element_type=jnp.float32,
        ).astype(jnp.bfloat16)

    def qrows(q):
        return pl.ds(pl.multiple_of(q * QROWS, QROWS), QROWS)

    def srows(a, j):
        return pl.ds(pl.multiple_of(a * QROWS + j * SUB, SUB), SUB)

    CW = pl.ds(0, HALF)
    CCW = pl.ds(HALF, HALF)

    def rdma(src, dst, ssem, rsem, dev):
        return pltpu.make_async_remote_copy(
            src_ref=src, dst_ref=dst, send_sem=ssem, recv_sem=rsem,
            device_id=(dev,), device_id_type=pl.DeviceIdType.MESH,
        )

    for s in range(3):
        cw = rdma(out_ref.at[qrows(lax.rem(pi - s + 4, 4)), CW],
                  stagA_r.at[s], sA_r.at[s], rA_r.at[s], plane_r)
        cc = rdma(out_ref.at[qrows(lax.rem(pi + s, 4)), CCW],
                  stagA_l.at[s], sA_l.at[s], rA_l.at[s], plane_l)
        cw.start()
        cc.start()
        cw.wait()
        qa = lax.rem(pi - s - 1 + 4, 4)
        out_ref[qrows(qa), :HALF] = out_ref[qrows(qa), :HALF] + stagA_r[s]
        cc.wait()
        qb = lax.rem(pi + s + 1, 4)
        out_ref[qrows(qb), HALF:] = out_ref[qrows(qb), HALF:] + stagA_l[s]

    a_r = lax.rem(pi + 1, 4)
    a_l = lax.rem(pi + 3, 4)

    for s in range(3):
        cw = rdma(out_ref.at[srows(a_r, lax.rem(zi - s + 4, 4)), CW],
                  stagB_r.at[s], sB_r.at[s], rB_r.at[s], z_r)
        cc = rdma(out_ref.at[srows(a_l, lax.rem(zi + s, 4)), CCW],
                  stagB_l.at[s], sB_l.at[s], rB_l.at[s], z_l)
        cw.start()
        cc.start()
        cw.wait()
        ja = lax.rem(zi - s - 1 + 4, 4)
        out_ref[srows(a_r, ja), :HALF] = (
            out_ref[srows(a_r, ja), :HALF] + stagB_r[s]
        )
        cc.wait()
        jb = lax.rem(zi + s + 1, 4)
        out_ref[srows(a_l, jb), HALF:] = (
            out_ref[srows(a_l, jb), HALF:] + stagB_l[s]
        )
    for t in range(3):
        src_r = out_ref.at[srows(a_r, lax.rem(zi + 1 - t + 4, 4)), CW]
        cw = rdma(src_r, src_r, sB_r.at[3 + t], rB_r.at[3 + t], z_r)
        src_l = out_ref.at[srows(a_l, lax.rem(zi + 3 + t, 4)), CCW]
        cc = rdma(src_l, src_l, sB_l.at[3 + t], rB_l.at[3 + t], z_l)
        cw.start()
        cc.start()
        cw.wait()
        cc.wait()

    for t in range(3):
        src_r = out_ref.at[qrows(lax.rem(pi + 1 - t + 4, 4)), CW]
        cw = rdma(src_r, src_r, sC_r.at[t], rC_r.at[t], plane_r)
        src_l = out_ref.at[qrows(lax.rem(pi + 3 + t, 4)), CCW]
        cc = rdma(src_l, src_l, sC_l.at[t], rC_l.at[t], plane_l)
        cw.start()
        cc.start()
        cw.wait()
        cc.wait()


def _ring_allreduce(ctx, wo_l):
    return pl.pallas_call(
        _allreduce_body,
        out_shape=jax.ShapeDtypeStruct((SQ, DM), jnp.bfloat16),
        in_specs=[
            pl.BlockSpec(memory_space=pltpu.VMEM),
            pl.BlockSpec(memory_space=pltpu.VMEM),
        ],
        out_specs=pl.BlockSpec(memory_space=pltpu.VMEM),
        scratch_shapes=[
            pltpu.VMEM((3, QROWS, HALF), jnp.bfloat16),
            pltpu.VMEM((3, QROWS, HALF), jnp.bfloat16),
            pltpu.VMEM((3, SUB, HALF), jnp.bfloat16),
            pltpu.VMEM((3, SUB, HALF), jnp.bfloat16),
            pltpu.SemaphoreType.DMA((3,)),
            pltpu.SemaphoreType.DMA((3,)),
            pltpu.SemaphoreType.DMA((3,)),
            pltpu.SemaphoreType.DMA((3,)),
            pltpu.SemaphoreType.DMA((6,)),
            pltpu.SemaphoreType.DMA((6,)),
            pltpu.SemaphoreType.DMA((6,)),
            pltpu.SemaphoreType.DMA((6,)),
            pltpu.SemaphoreType.DMA((3,)),
            pltpu.SemaphoreType.DMA((3,)),
            pltpu.SemaphoreType.DMA((3,)),
            pltpu.SemaphoreType.DMA((3,)),
        ],
        compiler_params=pltpu.CompilerParams(collective_id=0),
    )(ctx, wo_l)


SCALE = 0.08838834764831843
QBLK = 128
WIN = 384
NEG = -1e9


def _attn_body(x_ref, wq_ref, k_ref, v_ref, o_ref):
    qb = pl.program_id(1)

    xt = x_ref[pl.ds(pl.multiple_of(qb * QBLK, QBLK), QBLK), :]
    q = lax.dot_general(
        xt.astype(jnp.bfloat16), wq_ref[...], (((1,), (0,)), ((), ())),
        preferred_element_type=jnp.float32,
    ).astype(jnp.bfloat16)

    row = lax.broadcasted_iota(jnp.int32, (QBLK, 1), 0) + qb * QBLK

    @pl.when(qb == 0)
    def _dense():
        k = k_ref[...].astype(jnp.bfloat16)
        s = lax.dot_general(
            q, k, (((1,), (1,)), ((), ())), preferred_element_type=jnp.float32
        ) * SCALE
        ki = lax.broadcasted_iota(jnp.int32, (QBLK, SQ), 1)
        mask = (jnp.abs(row - ki) <= 128) | (ki < 32) | (row < 32)
        s = jnp.where(mask, s, NEG)
        m = jnp.max(s, axis=-1, keepdims=True)
        w = jnp.exp(s - m)
        denom = jnp.sum(w, axis=-1, keepdims=True)
        ctx = lax.dot_general(
            w.astype(jnp.bfloat16), v_ref[...].astype(jnp.bfloat16),
            (((1,), (0,)), ((), ())),
            preferred_element_type=jnp.float32,
        )
        o_ref[...] = (ctx / denom).astype(jnp.bfloat16)

    @pl.when(qb > 0)
    def _band():
        ws = pl.multiple_of(jnp.clip((qb - 1) * QBLK, 0, SQ - WIN), QBLK)
        kw = k_ref[pl.ds(ws, WIN), :].astype(jnp.bfloat16)
        vw = v_ref[pl.ds(ws, WIN), :].astype(jnp.bfloat16)
        k0 = k_ref[:QBLK, :].astype(jnp.bfloat16)
        v0 = v_ref[:QBLK, :].astype(jnp.bfloat16)

        sb = lax.dot_general(
            q, kw, (((1,), (1,)), ((), ())), preferred_element_type=jnp.float32
        ) * SCALE
        kib = lax.broadcasted_iota(jnp.int32, (QBLK, WIN), 1) + ws
        mb = (jnp.abs(row - kib) <= 128) | (kib < 32)
        sb = jnp.where(mb, sb, NEG)

        sg = lax.dot_general(
            q, k0, (((1,), (1,)), ((), ())), preferred_element_type=jnp.float32
        ) * SCALE
        kig = lax.broadcasted_iota(jnp.int32, (QBLK, QBLK), 1)
        mg = (kig < 32) & (qb >= 2)
        sg = jnp.where(mg, sg, NEG)

        m = jnp.maximum(
            jnp.max(sb, axis=-1, keepdims=True),
            jnp.max(sg, axis=-1, keepdims=True),
        )
        wb = jnp.exp(sb - m)
        wg = jnp.exp(sg - m)
        denom = jnp.sum(wb, axis=-1, keepdims=True) + jnp.sum(
            wg, axis=-1, keepdims=True
        )
        ctx = lax.dot_general(
            wb.astype(jnp.bfloat16), vw, (((1,), (0,)), ((), ())),
            preferred_element_type=jnp.float32,
        ) + lax.dot_general(
            wg.astype(jnp.bfloat16), v0, (((1,), (0,)), ((), ())),
            preferred_element_type=jnp.float32,
        )
        o_ref[...] = (ctx / denom).astype(jnp.bfloat16)


def _sparse_attn(x2d, wq_l, k2d, v2d, hq_per):
    return pl.pallas_call(
        _attn_body,
        grid=(hq_per, SQ // QBLK),
        out_shape=jax.ShapeDtypeStruct((SQ, DM), jnp.bfloat16),
        in_specs=[
            pl.BlockSpec((SQ, DM), lambda h, qb: (0, 0)),
            pl.BlockSpec((DM, 128), lambda h, qb: (0, h)),
            pl.BlockSpec((SQ, 128), lambda h, qb: (0, h)),
            pl.BlockSpec((SQ, 128), lambda h, qb: (0, h)),
        ],
        out_specs=pl.BlockSpec((QBLK, 128), lambda h, qb: (qb, h)),
    )(x2d, wq_l, k2d, v2d)


def kernel(x, Wq, K_ext, V_ext, Wo):
    my = lax.axis_index("i")
    sq = x.shape[1]
    hq_per = K_ext.shape[2]
    dh = K_ext.shape[3]
    dcols = hq_per * dh

    Wq_l = lax.dynamic_slice(Wq, (0, my * dcols), (Wq.shape[0], dcols))
    Wo_l = lax.dynamic_slice(Wo, (my * dcols, 0), (dcols, Wo.shape[1]))

    ctx = _sparse_attn(
        x[0],
        Wq_l.astype(jnp.bfloat16),
        K_ext[0].reshape(sq, dcols),
        V_ext[0].reshape(sq, dcols),
        hq_per,
    )

    out = _ring_allreduce(ctx, Wo_l.astype(jnp.bfloat16))
    return out.astype(jnp.float32)[None]


# device time: 171658 ns/iter; 1.2804x vs baseline; 1.2804x over previous
import jax
import jax.numpy as jnp
from jax import lax
from jax.experimental import pallas as pl
from jax.experimental.pallas import tpu as pltpu

N_DEV = 16
SQ = 2048
DM = 1024
CHUNK = SQ // N_DEV


HALF = DM // 2
QROWS = SQ // 4
SUB = SQ // 16


def _allreduce_body(
    ctx_ref, wo_ref, out_ref, stagA_r, stagA_l, stagB_r, stagB_l,
    sA_r, rA_r, sA_l, rA_l,
    sB_r, rB_r, sB_l, rB_l,
    sC_r, rC_r, sC_l, rC_l,
):
    my = lax.axis_index("i")
    zi = my // 4
    pi = lax.rem(my, 4)
    plane_r = zi * 4 + lax.rem(pi + 1, 4)
    plane_l = zi * 4 + lax.rem(pi + 3, 4)
    z_r = lax.rem(zi + 1, 4) * 4 + pi
    z_l = lax.rem(zi + 3, 4) * 4 + pi

    barrier = pltpu.get_barrier_semaphore()
    for nbr in (plane_l, plane_r, z_l, z_r):
        pl.semaphore_signal(
            barrier, inc=1, device_id=(nbr,), device_id_type=pl.DeviceIdType.MESH
        )
    pl.semaphore_wait(barrier, 4)

    for c in range(8):
        out_ref[pl.ds(c * 256, 256), :] = lax.dot_general(
            ctx_ref[pl.ds(c * 256, 256), :], wo_ref[...],
            (((1,), (0,)), ((), ())), preferred_element_type=jnp.float32,
        ).astype(jnp.bfloat16)

    def qrows(q):
        return pl.ds(pl.multiple_of(q * QROWS, QROWS), QROWS)

    def srows(a, j):
        return pl.ds(pl.multiple_of(a * QROWS + j * SUB, SUB), SUB)

    CW = pl.ds(0, HALF)
    CCW = pl.ds(HALF, HALF)

    def rdma(src, dst, ssem, rsem, dev):
        return pltpu.make_async_remote_copy(
            src_ref=src, dst_ref=dst, send_sem=ssem, recv_sem=rsem,
            device_id=(dev,), device_id_type=pl.DeviceIdType.MESH,
        )

    for s in range(3):
        cw = rdma(out_ref.at[qrows(lax.rem(pi - s + 4, 4)), CW],
                  stagA_r.at[s], sA_r.at[s], rA_r.at[s], plane_r)
        cc = rdma(out_ref.at[qrows(lax.rem(pi + s, 4)), CCW],
                  stagA_l.at[s], sA_l.at[s], rA_l.at[s], plane_l)
        cw.start()
        cc.start()
        cw.wait()
        qa = lax.rem(pi - s - 1 + 4, 4)
        out_ref[qrows(qa), :HALF] = out_ref[qrows(qa), :HALF] + stagA_r[s]
        cc.wait()
        qb = lax.rem(pi + s + 1, 4)
        out_ref[qrows(qb), HALF:] = out_ref[qrows(qb), HALF:] + stagA_l[s]

    a_r = lax.rem(pi + 1, 4)
    a_l = lax.rem(pi + 3, 4)

    for s in range(3):
        cw = rdma(out_ref.at[srows(a_r, lax.rem(zi - s + 4, 4)), CW],
                  stagB_r.at[s], sB_r.at[s], rB_r.at[s], z_r)
        cc = rdma(out_ref.at[srows(a_l, lax.rem(zi + s, 4)), CCW],
                  stagB_l.at[s], sB_l.at[s], rB_l.at[s], z_l)
        cw.start()
        cc.start()
        cw.wait()
        ja = lax.rem(zi - s - 1 + 4, 4)
        out_ref[srows(a_r, ja), :HALF] = (
            out_ref[srows(a_r, ja), :HALF] + stagB_r[s]
        )
        cc.wait()
        jb = lax.rem(zi + s + 1, 4)
        out_ref[srows(a_l, jb), HALF:] = (
            out_ref[srows(a_l, jb), HALF:] + stagB_l[s]
        )
    for t in range(3):
        src_r = out_ref.at[srows(a_r, lax.rem(zi + 1 - t + 4, 4)), CW]
        cw = rdma(src_r, src_r, sB_r.at[3 + t], rB_r.at[3 + t], z_r)
        src_l = out_ref.at[srows(a_l, lax.rem(zi + 3 + t, 4)), CCW]
        cc = rdma(src_l, src_l, sB_l.at[3 + t], rB_l.at[3 + t], z_l)
        cw.start()
        cc.start()
        cw.wait()
        cc.wait()

    for t in range(3):
        src_r = out_ref.at[qrows(lax.rem(pi + 1 - t + 4, 4)), CW]
        cw = rdma(src_r, src_r, sC_r.at[t], rC_r.at[t], plane_r)
        src_l = out_ref.at[qrows(lax.rem(pi + 3 + t, 4)), CCW]
        cc = rdma(src_l, src_l, sC_l.at[t], rC_l.at[t], plane_l)
        cw.start()
        cc.start()
        cw.wait()
        cc.wait()


def _ring_allreduce(ctx, wo_l):
    return pl.pallas_call(
        _allreduce_body,
        out_shape=jax.ShapeDtypeStruct((SQ, DM), jnp.bfloat16),
        in_specs=[
            pl.BlockSpec(memory_space=pltpu.VMEM),
            pl.BlockSpec(memory_space=pltpu.VMEM),
        ],
        out_specs=pl.BlockSpec(memory_space=pltpu.VMEM),
        scratch_shapes=[
            pltpu.VMEM((3, QROWS, HALF), jnp.bfloat16),
            pltpu.VMEM((3, QROWS, HALF), jnp.bfloat16),
            pltpu.VMEM((3, SUB, HALF), jnp.bfloat16),
            pltpu.VMEM((3, SUB, HALF), jnp.bfloat16),
            pltpu.SemaphoreType.DMA((3,)),
            pltpu.SemaphoreType.DMA((3,)),
            pltpu.SemaphoreType.DMA((3,)),
            pltpu.SemaphoreType.DMA((3,)),
            pltpu.SemaphoreType.DMA((6,)),
            pltpu.SemaphoreType.DMA((6,)),
            pltpu.SemaphoreType.DMA((6,)),
            pltpu.SemaphoreType.DMA((6,)),
            pltpu.SemaphoreType.DMA((3,)),
            pltpu.SemaphoreType.DMA((3,)),
            pltpu.SemaphoreType.DMA((3,)),
            pltpu.SemaphoreType.DMA((3,)),
        ],
        compiler_params=pltpu.CompilerParams(collective_id=0),
    )(ctx, wo_l)


SCALE = 0.08838834764831843
QBLK = 128
WIN = 384
NEG = -1e9


def _attn_body(x_ref, wq_ref, k_ref, v_ref, o_ref):
    qb = pl.program_id(0)
    n_heads = DM // 128

    xt = x_ref[pl.ds(pl.multiple_of(qb * QBLK, QBLK), QBLK), :]
    q_all = lax.dot_general(
        xt.astype(jnp.bfloat16), wq_ref[...], (((1,), (0,)), ((), ())),
        preferred_element_type=jnp.float32,
    ).astype(jnp.bfloat16)

    row = lax.broadcasted_iota(jnp.int32, (QBLK, 1), 0) + qb * QBLK

    @pl.when(qb == 0)
    def _dense():
        ki = lax.broadcasted_iota(jnp.int32, (QBLK, SQ), 1)
        mask = (jnp.abs(row - ki) <= 128) | (ki < 32) | (row < 32)
        for h in range(n_heads):
            hc = slice(h * 128, (h + 1) * 128)
            q = q_all[:, hc]
            k = k_ref[:, hc].astype(jnp.bfloat16)
            s = lax.dot_general(
                q, k, (((1,), (1,)), ((), ())),
                preferred_element_type=jnp.float32,
            ) * SCALE
            s = jnp.where(mask, s, NEG)
            m = jnp.max(s, axis=-1, keepdims=True)
            w = jnp.exp(s - m)
            denom = jnp.sum(w, axis=-1, keepdims=True)
            ctx = lax.dot_general(
                w.astype(jnp.bfloat16), v_ref[:, hc].astype(jnp.bfloat16),
                (((1,), (0,)), ((), ())),
                preferred_element_type=jnp.float32,
            )
            o_ref[:, hc] = (ctx / denom).astype(jnp.bfloat16)

    @pl.when(qb > 0)
    def _band():
        ws = pl.multiple_of(jnp.clip((qb - 1) * QBLK, 0, SQ - WIN), QBLK)
        kib = lax.broadcasted_iota(jnp.int32, (QBLK, WIN), 1) + ws
        mb = (jnp.abs(row - kib) <= 128) | (kib < 32)
        kig = lax.broadcasted_iota(jnp.int32, (QBLK, QBLK), 1)
        mg = (kig < 32) & (qb >= 2)
        for h in range(n_heads):
            hc = slice(h * 128, (h + 1) * 128)
            q = q_all[:, hc]
            kw = k_ref[pl.ds(ws, WIN), hc].astype(jnp.bfloat16)
            vw = v_ref[pl.ds(ws, WIN), hc].astype(jnp.bfloat16)
            k0 = k_ref[:QBLK, hc].astype(jnp.bfloat16)
            v0 = v_ref[:QBLK, hc].astype(jnp.bfloat16)

            sb = lax.dot_general(
                q, kw, (((1,), (1,)), ((), ())),
                preferred_element_type=jnp.float32,
            ) * SCALE
            sb = jnp.where(mb, sb, NEG)
            sg = lax.dot_general(
                q, k0, (((1,), (1,)), ((), ())),
                preferred_element_type=jnp.float32,
            ) * SCALE
            sg = jnp.where(mg, sg, NEG)

            m = jnp.maximum(
                jnp.max(sb, axis=-1, keepdims=True),
                jnp.max(sg, axis=-1, keepdims=True),
            )
            wb = jnp.exp(sb - m)
            wg = jnp.exp(sg - m)
            denom = jnp.sum(wb, axis=-1, keepdims=True) + jnp.sum(
                wg, axis=-1, keepdims=True
            )
            ctx = lax.dot_general(
                wb.astype(jnp.bfloat16), vw, (((1,), (0,)), ((), ())),
                preferred_element_type=jnp.float32,
            ) + lax.dot_general(
                wg.astype(jnp.bfloat16), v0, (((1,), (0,)), ((), ())),
                preferred_element_type=jnp.float32,
            )
            o_ref[:, hc] = (ctx / denom).astype(jnp.bfloat16)


def _sparse_attn(x2d, wq_l, k2d, v2d, hq_per):
    del hq_per
    return pl.pallas_call(
        _attn_body,
        grid=(SQ // QBLK,),
        out_shape=jax.ShapeDtypeStruct((SQ, DM), jnp.bfloat16),
        in_specs=[
            pl.BlockSpec((SQ, DM), lambda qb: (0, 0)),
            pl.BlockSpec((DM, DM), lambda qb: (0, 0)),
            pl.BlockSpec((SQ, DM), lambda qb: (0, 0)),
            pl.BlockSpec((SQ, DM), lambda qb: (0, 0)),
        ],
        out_specs=pl.BlockSpec((QBLK, DM), lambda qb: (qb, 0)),
    )(x2d, wq_l, k2d, v2d)


def kernel(x, Wq, K_ext, V_ext, Wo):
    my = lax.axis_index("i")
    sq = x.shape[1]
    hq_per = K_ext.shape[2]
    dh = K_ext.shape[3]
    dcols = hq_per * dh

    Wq_l = lax.dynamic_slice(Wq, (0, my * dcols), (Wq.shape[0], dcols))
    Wo_l = lax.dynamic_slice(Wo, (my * dcols, 0), (dcols, Wo.shape[1]))

    ctx = _sparse_attn(
        x[0],
        Wq_l.astype(jnp.bfloat16),
        K_ext[0].reshape(sq, dcols),
        V_ext[0].reshape(sq, dcols),
        hq_per,
    )

    out = _ring_allreduce(ctx, Wo_l.astype(jnp.bfloat16))
    return out.astype(jnp.float32)[None]


# device time: 162648 ns/iter; 1.3514x vs baseline; 1.0554x over previous
import jax
import jax.numpy as jnp
from jax import lax
from jax.experimental import pallas as pl
from jax.experimental.pallas import tpu as pltpu

N_DEV = 16
SQ = 2048
DM = 1024
CHUNK = SQ // N_DEV


HALF = DM // 2
QROWS = SQ // 4
SUB = SQ // 16


def _allreduce_body(
    ctx_ref, wo_ref, out_ref, stagA_r, stagA_l, stagB_r, stagB_l,
    sA_r, rA_r, sA_l, rA_l,
    sB_r, rB_r, sB_l, rB_l,
    sC_r, rC_r, sC_l, rC_l,
):
    my = lax.axis_index("i")
    zi = my // 4
    pi = lax.rem(my, 4)
    plane_r = zi * 4 + lax.rem(pi + 1, 4)
    plane_l = zi * 4 + lax.rem(pi + 3, 4)
    z_r = lax.rem(zi + 1, 4) * 4 + pi
    z_l = lax.rem(zi + 3, 4) * 4 + pi

    barrier = pltpu.get_barrier_semaphore()
    for nbr in (plane_l, plane_r, z_l, z_r):
        pl.semaphore_signal(
            barrier, inc=1, device_id=(nbr,), device_id_type=pl.DeviceIdType.MESH
        )
    pl.semaphore_wait(barrier, 4)

    for c in range(8):
        out_ref[pl.ds(c * 256, 256), :] = lax.dot_general(
            ctx_ref[pl.ds(c * 256, 256), :], wo_ref[...],
            (((1,), (0,)), ((), ())), preferred_element_type=jnp.float32,
        ).astype(jnp.bfloat16)

    def qrows(q):
        return pl.ds(pl.multiple_of(q * QROWS, QROWS), QROWS)

    def srows(a, j):
        return pl.ds(pl.multiple_of(a * QROWS + j * SUB, SUB), SUB)

    CW = pl.ds(0, HALF)
    CCW = pl.ds(HALF, HALF)

    def rdma(src, dst, ssem, rsem, dev):
        return pltpu.make_async_remote_copy(
            src_ref=src, dst_ref=dst, send_sem=ssem, recv_sem=rsem,
            device_id=(dev,), device_id_type=pl.DeviceIdType.MESH,
        )

    HR = QROWS // 2

    def hrows(q, half):
        return pl.ds(pl.multiple_of(q * QROWS + half * HR, HR), HR)

    cw_d = [None, None]
    cc_d = [None, None]
    for s in range(3):
        for half in range(2):
            if s > 0:
                idx0 = (s - 1) * 2 + half
                cw_d[half].wait()
                qa = lax.rem(pi - s + 4, 4)
                out_ref[hrows(qa, half), :HALF] = (
                    out_ref[hrows(qa, half), :HALF] + stagA_r[idx0]
                )
                cc_d[half].wait()
                qb = lax.rem(pi + s, 4)
                out_ref[hrows(qb, half), HALF:] = (
                    out_ref[hrows(qb, half), HALF:] + stagA_l[idx0]
                )
            idx = s * 2 + half
            cw_d[half] = rdma(
                out_ref.at[hrows(lax.rem(pi - s + 4, 4), half), CW],
                stagA_r.at[idx], sA_r.at[idx], rA_r.at[idx], plane_r)
            cc_d[half] = rdma(
                out_ref.at[hrows(lax.rem(pi + s, 4), half), CCW],
                stagA_l.at[idx], sA_l.at[idx], rA_l.at[idx], plane_l)
            cw_d[half].start()
            cc_d[half].start()
    for half in range(2):
        idx0 = 4 + half
        cw_d[half].wait()
        qa = lax.rem(pi + 1, 4)
        out_ref[hrows(qa, half), :HALF] = (
            out_ref[hrows(qa, half), :HALF] + stagA_r[idx0]
        )
        cc_d[half].wait()
        qb = lax.rem(pi + 3, 4)
        out_ref[hrows(qb, half), HALF:] = (
            out_ref[hrows(qb, half), HALF:] + stagA_l[idx0]
        )

    a_r = lax.rem(pi + 1, 4)
    a_l = lax.rem(pi + 3, 4)

    bw = bc = None
    for s in range(3):
        if s > 0:
            bw.wait()
            ja = lax.rem(zi - s + 4, 4)
            out_ref[srows(a_r, ja), :HALF] = (
                out_ref[srows(a_r, ja), :HALF] + stagB_r[s - 1]
            )
        bw = rdma(out_ref.at[srows(a_r, lax.rem(zi - s + 4, 4)), CW],
                  stagB_r.at[s], sB_r.at[s], rB_r.at[s], z_r)
        bw.start()
        if s > 0:
            bc.wait()
            jb = lax.rem(zi + s, 4)
            out_ref[srows(a_l, jb), HALF:] = (
                out_ref[srows(a_l, jb), HALF:] + stagB_l[s - 1]
            )
        bc = rdma(out_ref.at[srows(a_l, lax.rem(zi + s, 4)), CCW],
                  stagB_l.at[s], sB_l.at[s], rB_l.at[s], z_l)
        bc.start()
    bw.wait()
    ja = lax.rem(zi + 1, 4)
    out_ref[srows(a_r, ja), :HALF] = (
        out_ref[srows(a_r, ja), :HALF] + stagB_r[2]
    )
    bc.wait()
    jb = lax.rem(zi + 3, 4)
    out_ref[srows(a_l, jb), HALF:] = (
        out_ref[srows(a_l, jb), HALF:] + stagB_l[2]
    )

    for t in range(3):
        if t > 0:
            bw.wait()
            bc.wait()
        src_r = out_ref.at[srows(a_r, lax.rem(zi + 1 - t + 4, 4)), CW]
        bw = rdma(src_r, src_r, sB_r.at[3 + t], rB_r.at[3 + t], z_r)
        bw.start()
        src_l = out_ref.at[srows(a_l, lax.rem(zi + 3 + t, 4)), CCW]
        bc = rdma(src_l, src_l, sB_l.at[3 + t], rB_l.at[3 + t], z_l)
        bc.start()
    bw.wait()
    bc.wait()

    for t in range(3):
        for half in range(2):
            if t > 0:
                cw_d[half].wait()
                cc_d[half].wait()
            idx = t * 2 + half
            src_r = out_ref.at[hrows(lax.rem(pi + 1 - t + 4, 4), half), CW]
            cw_d[half] = rdma(src_r, src_r, sC_r.at[idx], rC_r.at[idx], plane_r)
            src_l = out_ref.at[hrows(lax.rem(pi + 3 + t, 4), half), CCW]
            cc_d[half] = rdma(src_l, src_l, sC_l.at[idx], rC_l.at[idx], plane_l)
            cw_d[half].start()
            cc_d[half].start()
    for half in range(2):
        cw_d[half].wait()
        cc_d[half].wait()


def _ring_allreduce(ctx, wo_l):
    return pl.pallas_call(
        _allreduce_body,
        out_shape=jax.ShapeDtypeStruct((SQ, DM), jnp.bfloat16),
        in_specs=[
            pl.BlockSpec(memory_space=pltpu.VMEM),
            pl.BlockSpec(memory_space=pltpu.VMEM),
        ],
        out_specs=pl.BlockSpec(memory_space=pltpu.VMEM),
        scratch_shapes=[
            pltpu.VMEM((6, QROWS // 2, HALF), jnp.bfloat16),
            pltpu.VMEM((6, QROWS // 2, HALF), jnp.bfloat16),
            pltpu.VMEM((3, SUB, HALF), jnp.bfloat16),
            pltpu.VMEM((3, SUB, HALF), jnp.bfloat16),
            pltpu.SemaphoreType.DMA((6,)),
            pltpu.SemaphoreType.DMA((6,)),
            pltpu.SemaphoreType.DMA((6,)),
            pltpu.SemaphoreType.DMA((6,)),
            pltpu.SemaphoreType.DMA((6,)),
            pltpu.SemaphoreType.DMA((6,)),
            pltpu.SemaphoreType.DMA((6,)),
            pltpu.SemaphoreType.DMA((6,)),
            pltpu.SemaphoreType.DMA((6,)),
            pltpu.SemaphoreType.DMA((6,)),
            pltpu.SemaphoreType.DMA((6,)),
            pltpu.SemaphoreType.DMA((6,)),
        ],
        compiler_params=pltpu.CompilerParams(collective_id=0),
    )(ctx, wo_l)


SCALE = 0.08838834764831843
QBLK = 128
WIN = 384
NEG = -1e9


def _attn_body(x_ref, wq_ref, k_ref, v_ref, o_ref):
    qb = pl.program_id(0)
    n_heads = DM // 128

    xt = x_ref[pl.ds(pl.multiple_of(qb * QBLK, QBLK), QBLK), :]
    q_all = lax.dot_general(
        xt.astype(jnp.bfloat16), wq_ref[...], (((1,), (0,)), ((), ())),
        preferred_element_type=jnp.float32,
    ).astype(jnp.bfloat16)

    row = lax.broadcasted_iota(jnp.int32, (QBLK, 1), 0) + qb * QBLK

    @pl.when(qb == 0)
    def _dense():
        ki = lax.broadcasted_iota(jnp.int32, (QBLK, SQ), 1)
        mask = (jnp.abs(row - ki) <= 128) | (ki < 32) | (row < 32)
        for h in range(n_heads):
            hc = slice(h * 128, (h + 1) * 128)
            q = q_all[:, hc]
            k = k_ref[:, hc].astype(jnp.bfloat16)
            s = lax.dot_general(
                q, k, (((1,), (1,)), ((), ())),
                preferred_element_type=jnp.float32,
            ) * SCALE
            s = jnp.where(mask, s, NEG)
            m = jnp.max(s, axis=-1, keepdims=True)
            w = jnp.exp(s - m)
            denom = jnp.sum(w, axis=-1, keepdims=True)
            ctx = lax.dot_general(
                w.astype(jnp.bfloat16), v_ref[:, hc].astype(jnp.bfloat16),
                (((1,), (0,)), ((), ())),
                preferred_element_type=jnp.float32,
            )
            o_ref[:, hc] = (ctx / denom).astype(jnp.bfloat16)

    @pl.when(qb > 0)
    def _band():
        ws = pl.multiple_of(jnp.clip((qb - 1) * QBLK, 0, SQ - WIN), QBLK)
        kib = lax.broadcasted_iota(jnp.int32, (QBLK, WIN), 1) + ws
        mb = (jnp.abs(row - kib) <= 128) | (kib < 32)
        kig = lax.broadcasted_iota(jnp.int32, (QBLK, QBLK), 1)
        mg = (kig < 32) & (qb >= 2)
        for h in range(n_heads):
            hc = slice(h * 128, (h + 1) * 128)
            q = q_all[:, hc]
            kw = k_ref[pl.ds(ws, WIN), hc].astype(jnp.bfloat16)
            vw = v_ref[pl.ds(ws, WIN), hc].astype(jnp.bfloat16)
            k0 = k_ref[:QBLK, hc].astype(jnp.bfloat16)
            v0 = v_ref[:QBLK, hc].astype(jnp.bfloat16)

            sb = lax.dot_general(
                q, kw, (((1,), (1,)), ((), ())),
                preferred_element_type=jnp.float32,
            ) * SCALE
            sb = jnp.where(mb, sb, NEG)
            sg = lax.dot_general(
                q, k0, (((1,), (1,)), ((), ())),
                preferred_element_type=jnp.float32,
            ) * SCALE
            sg = jnp.where(mg, sg, NEG)

            m = jnp.maximum(
                jnp.max(sb, axis=-1, keepdims=True),
                jnp.max(sg, axis=-1, keepdims=True),
            )
            wb = jnp.exp(sb - m)
            wg = jnp.exp(sg - m)
            denom = jnp.sum(wb, axis=-1, keepdims=True) + jnp.sum(
                wg, axis=-1, keepdims=True
            )
            ctx = lax.dot_general(
                wb.astype(jnp.bfloat16), vw, (((1,), (0,)), ((), ())),
                preferred_element_type=jnp.float32,
            ) + lax.dot_general(
                wg.astype(jnp.bfloat16), v0, (((1,), (0,)), ((), ())),
                preferred_element_type=jnp.float32,
            )
            o_ref[:, hc] = (ctx / denom).astype(jnp.bfloat16)


def _sparse_attn(x2d, wq_l, k2d, v2d, hq_per):
    del hq_per
    return pl.pallas_call(
        _attn_body,
        grid=(SQ // QBLK,),
        out_shape=jax.ShapeDtypeStruct((SQ, DM), jnp.bfloat16),
        in_specs=[
            pl.BlockSpec((SQ, DM), lambda qb: (0, 0)),
            pl.BlockSpec((DM, DM), lambda qb: (0, 0)),
            pl.BlockSpec((SQ, DM), lambda qb: (0, 0)),
            pl.BlockSpec((SQ, DM), lambda qb: (0, 0)),
        ],
        out_specs=pl.BlockSpec((QBLK, DM), lambda qb: (qb, 0)),
    )(x2d, wq_l, k2d, v2d)


def kernel(x, Wq, K_ext, V_ext, Wo):
    my = lax.axis_index("i")
    sq = x.shape[1]
    hq_per = K_ext.shape[2]
    dh = K_ext.shape[3]
    dcols = hq_per * dh

    Wq_l = lax.dynamic_slice(Wq, (0, my * dcols), (Wq.shape[0], dcols))
    Wo_l = lax.dynamic_slice(Wo, (my * dcols, 0), (dcols, Wo.shape[1]))

    ctx = _sparse_attn(
        x[0],
        Wq_l.astype(jnp.bfloat16),
        K_ext[0].reshape(sq, dcols),
        V_ext[0].reshape(sq, dcols),
        hq_per,
    )

    out = _ring_allreduce(ctx, Wo_l.astype(jnp.bfloat16))
    return out.astype(jnp.float32)[None]


# device time: 160152 ns/iter; 1.3724x vs baseline; 1.0156x over previous
import jax
import jax.numpy as jnp
from jax import lax
from jax.experimental import pallas as pl
from jax.experimental.pallas import tpu as pltpu

N_DEV = 16
SQ = 2048
DM = 1024
CHUNK = SQ // N_DEV


HALF = DM // 2
QROWS = SQ // 4
SUB = SQ // 16


def _allreduce_body(
    ctx_ref, wo_ref, out_ref, stagA_r, stagA_l, stagB_r, stagB_l,
    sA_r, rA_r, sA_l, rA_l,
    sB_r, rB_r, sB_l, rB_l,
    sC_r, rC_r, sC_l, rC_l,
):
    my = lax.axis_index("i")
    zi = my // 4
    pi = lax.rem(my, 4)
    plane_r = zi * 4 + lax.rem(pi + 1, 4)
    plane_l = zi * 4 + lax.rem(pi + 3, 4)
    z_r = lax.rem(zi + 1, 4) * 4 + pi
    z_l = lax.rem(zi + 3, 4) * 4 + pi

    barrier = pltpu.get_barrier_semaphore()
    for nbr in (plane_l, plane_r, z_l, z_r):
        pl.semaphore_signal(
            barrier, inc=1, device_id=(nbr,), device_id_type=pl.DeviceIdType.MESH
        )
    pl.semaphore_wait(barrier, 4)

    for c in range(8):
        out_ref[pl.ds(c * 256, 256), :] = lax.dot_general(
            ctx_ref[pl.ds(c * 256, 256), :], wo_ref[...],
            (((1,), (0,)), ((), ())), preferred_element_type=jnp.float32,
        ).astype(jnp.bfloat16)

    def qrows(q):
        return pl.ds(pl.multiple_of(q * QROWS, QROWS), QROWS)

    def srows(a, j):
        return pl.ds(pl.multiple_of(a * QROWS + j * SUB, SUB), SUB)

    CW = pl.ds(0, HALF)
    CCW = pl.ds(HALF, HALF)

    def rdma(src, dst, ssem, rsem, dev):
        return pltpu.make_async_remote_copy(
            src_ref=src, dst_ref=dst, send_sem=ssem, recv_sem=rsem,
            device_id=(dev,), device_id_type=pl.DeviceIdType.MESH,
        )

    HR = QROWS // 2

    def hrows(q, half):
        return pl.ds(pl.multiple_of(q * QROWS + half * HR, HR), HR)

    cw_d = [None, None]
    cc_d = [None, None]
    for s in range(3):
        for half in range(2):
            if s > 0:
                idx0 = (s - 1) * 2 + half
                cw_d[half].wait()
                qa = lax.rem(pi - s + 4, 4)
                out_ref[hrows(qa, half), :HALF] = (
                    out_ref[hrows(qa, half), :HALF] + stagA_r[idx0]
                )
                cc_d[half].wait()
                qb = lax.rem(pi + s, 4)
                out_ref[hrows(qb, half), HALF:] = (
                    out_ref[hrows(qb, half), HALF:] + stagA_l[idx0]
                )
            idx = s * 2 + half
            cw_d[half] = rdma(
                out_ref.at[hrows(lax.rem(pi - s + 4, 4), half), CW],
                stagA_r.at[idx], sA_r.at[idx], rA_r.at[idx], plane_r)
            cc_d[half] = rdma(
                out_ref.at[hrows(lax.rem(pi + s, 4), half), CCW],
                stagA_l.at[idx], sA_l.at[idx], rA_l.at[idx], plane_l)
            cw_d[half].start()
            cc_d[half].start()
    for half in range(2):
        idx0 = 4 + half
        cw_d[half].wait()
        qa = lax.rem(pi + 1, 4)
        out_ref[hrows(qa, half), :HALF] = (
            out_ref[hrows(qa, half), :HALF] + stagA_r[idx0]
        )
        cc_d[half].wait()
        qb = lax.rem(pi + 3, 4)
        out_ref[hrows(qb, half), HALF:] = (
            out_ref[hrows(qb, half), HALF:] + stagA_l[idx0]
        )

    a_r = lax.rem(pi + 1, 4)
    a_l = lax.rem(pi + 3, 4)

    bw = bc = None
    for s in range(3):
        if s > 0:
            bw.wait()
            ja = lax.rem(zi - s + 4, 4)
            out_ref[srows(a_r, ja), :HALF] = (
                out_ref[srows(a_r, ja), :HALF] + stagB_r[s - 1]
            )
        bw = rdma(out_ref.at[srows(a_r, lax.rem(zi - s + 4, 4)), CW],
                  stagB_r.at[s], sB_r.at[s], rB_r.at[s], z_r)
        bw.start()
        if s > 0:
            bc.wait()
            jb = lax.rem(zi + s, 4)
            out_ref[srows(a_l, jb), HALF:] = (
                out_ref[srows(a_l, jb), HALF:] + stagB_l[s - 1]
            )
        bc = rdma(out_ref.at[srows(a_l, lax.rem(zi + s, 4)), CCW],
                  stagB_l.at[s], sB_l.at[s], rB_l.at[s], z_l)
        bc.start()
    bw.wait()
    ja = lax.rem(zi + 1, 4)
    out_ref[srows(a_r, ja), :HALF] = (
        out_ref[srows(a_r, ja), :HALF] + stagB_r[2]
    )
    bc.wait()
    jb = lax.rem(zi + 3, 4)
    out_ref[srows(a_l, jb), HALF:] = (
        out_ref[srows(a_l, jb), HALF:] + stagB_l[2]
    )

    for t in range(3):
        if t > 0:
            bw.wait()
            bc.wait()
        src_r = out_ref.at[srows(a_r, lax.rem(zi + 1 - t + 4, 4)), CW]
        bw = rdma(src_r, src_r, sB_r.at[3 + t], rB_r.at[3 + t], z_r)
        bw.start()
        src_l = out_ref.at[srows(a_l, lax.rem(zi + 3 + t, 4)), CCW]
        bc = rdma(src_l, src_l, sB_l.at[3 + t], rB_l.at[3 + t], z_l)
        bc.start()
    bw.wait()
    bc.wait()

    for t in range(3):
        for half in range(2):
            if t > 0:
                cw_d[half].wait()
                cc_d[half].wait()
            idx = t * 2 + half
            src_r = out_ref.at[hrows(lax.rem(pi + 1 - t + 4, 4), half), CW]
            cw_d[half] = rdma(src_r, src_r, sC_r.at[idx], rC_r.at[idx], plane_r)
            src_l = out_ref.at[hrows(lax.rem(pi + 3 + t, 4), half), CCW]
            cc_d[half] = rdma(src_l, src_l, sC_l.at[idx], rC_l.at[idx], plane_l)
            cw_d[half].start()
            cc_d[half].start()
    for half in range(2):
        cw_d[half].wait()
        cc_d[half].wait()


def _ring_allreduce(ctx, wo_l):
    return pl.pallas_call(
        _allreduce_body,
        out_shape=jax.ShapeDtypeStruct((SQ, DM), jnp.bfloat16),
        in_specs=[
            pl.BlockSpec(memory_space=pltpu.VMEM),
            pl.BlockSpec(memory_space=pltpu.VMEM),
        ],
        out_specs=pl.BlockSpec(memory_space=pltpu.VMEM),
        scratch_shapes=[
            pltpu.VMEM((6, QROWS // 2, HALF), jnp.bfloat16),
            pltpu.VMEM((6, QROWS // 2, HALF), jnp.bfloat16),
            pltpu.VMEM((3, SUB, HALF), jnp.bfloat16),
            pltpu.VMEM((3, SUB, HALF), jnp.bfloat16),
            pltpu.SemaphoreType.DMA((6,)),
            pltpu.SemaphoreType.DMA((6,)),
            pltpu.SemaphoreType.DMA((6,)),
            pltpu.SemaphoreType.DMA((6,)),
            pltpu.SemaphoreType.DMA((6,)),
            pltpu.SemaphoreType.DMA((6,)),
            pltpu.SemaphoreType.DMA((6,)),
            pltpu.SemaphoreType.DMA((6,)),
            pltpu.SemaphoreType.DMA((6,)),
            pltpu.SemaphoreType.DMA((6,)),
            pltpu.SemaphoreType.DMA((6,)),
            pltpu.SemaphoreType.DMA((6,)),
        ],
        compiler_params=pltpu.CompilerParams(collective_id=0),
    )(ctx, wo_l)


SCALE = 0.08838834764831843
QBLK = 128
WIN = 384
NEG = -1e9


def _attn_body(x_ref, wq_ref, k_ref, v_ref, o_ref):
    qb = pl.program_id(0)
    n_heads = DM // 128

    xt = x_ref[pl.ds(pl.multiple_of(qb * QBLK, QBLK), QBLK), :]
    q_all = lax.dot_general(
        xt.astype(jnp.bfloat16), wq_ref[...], (((1,), (0,)), ((), ())),
        preferred_element_type=jnp.float32,
    ).astype(jnp.bfloat16)

    row = lax.broadcasted_iota(jnp.int32, (QBLK, 1), 0) + qb * QBLK

    @pl.when(qb == 0)
    def _dense():
        ki = lax.broadcasted_iota(jnp.int32, (QBLK, SQ), 1)
        mask = (jnp.abs(row - ki) <= 128) | (ki < 32) | (row < 32)
        for h in range(n_heads):
            hc = slice(h * 128, (h + 1) * 128)
            q = q_all[:, hc]
            k = k_ref[:, hc].astype(jnp.bfloat16)
            s = lax.dot_general(
                q, k, (((1,), (1,)), ((), ())),
                preferred_element_type=jnp.float32,
            ) * SCALE
            s = jnp.where(mask, s, NEG)
            m = jnp.max(s, axis=-1, keepdims=True)
            w = jnp.exp(s - m)
            denom = jnp.sum(w, axis=-1, keepdims=True)
            ctx = lax.dot_general(
                w.astype(jnp.bfloat16), v_ref[:, hc].astype(jnp.bfloat16),
                (((1,), (0,)), ((), ())),
                preferred_element_type=jnp.float32,
            )
            o_ref[:, hc] = (ctx / denom).astype(jnp.bfloat16)

    @pl.when(qb > 0)
    def _band():
        ws = pl.multiple_of(jnp.clip((qb - 1) * QBLK, 0, SQ - WIN), QBLK)
        kib = lax.broadcasted_iota(jnp.int32, (QBLK, WIN), 1) + ws
        mb = (jnp.abs(row - kib) <= 128) | (kib < 32)
        kig = lax.broadcasted_iota(jnp.int32, (QBLK, QBLK), 1)
        mg = (kig < 32) & (qb >= 2)
        for h in range(n_heads):
            hc = slice(h * 128, (h + 1) * 128)
            q = q_all[:, hc]
            kw = k_ref[pl.ds(ws, WIN), hc].astype(jnp.bfloat16)
            vw = v_ref[pl.ds(ws, WIN), hc].astype(jnp.bfloat16)
            k0 = k_ref[:QBLK, hc].astype(jnp.bfloat16)
            v0 = v_ref[:QBLK, hc].astype(jnp.bfloat16)

            sb = lax.dot_general(
                q, kw, (((1,), (1,)), ((), ())),
                preferred_element_type=jnp.float32,
            ) * SCALE
            sb = jnp.where(mb, sb, NEG)
            sg = lax.dot_general(
                q, k0, (((1,), (1,)), ((), ())),
                preferred_element_type=jnp.float32,
            ) * SCALE
            sg = jnp.where(mg, sg, NEG)

            m = jnp.maximum(
                jnp.max(sb, axis=-1, keepdims=True),
                jnp.max(sg, axis=-1, keepdims=True),
            )
            wb = jnp.exp(sb - m)
            wg = jnp.exp(sg - m)
            denom = jnp.sum(wb, axis=-1, keepdims=True) + jnp.sum(
                wg, axis=-1, keepdims=True
            )
            ctx = lax.dot_general(
                wb.astype(jnp.bfloat16), vw, (((1,), (0,)), ((), ())),
                preferred_element_type=jnp.float32,
            ) + lax.dot_general(
                wg.astype(jnp.bfloat16), v0, (((1,), (0,)), ((), ())),
                preferred_element_type=jnp.float32,
            )
            o_ref[:, hc] = (ctx / denom).astype(jnp.bfloat16)


def _sparse_attn(x2d, wq_l, k2d, v2d, hq_per):
    del hq_per
    return pl.pallas_call(
        _attn_body,
        grid=(SQ // QBLK,),
        out_shape=jax.ShapeDtypeStruct((SQ, DM), jnp.bfloat16),
        in_specs=[
            pl.BlockSpec((SQ, DM), lambda qb: (0, 0)),
            pl.BlockSpec((DM, DM), lambda qb: (0, 0)),
            pl.BlockSpec((SQ, DM), lambda qb: (0, 0)),
            pl.BlockSpec((SQ, DM), lambda qb: (0, 0)),
        ],
        out_specs=pl.BlockSpec((QBLK, DM), lambda qb: (qb, 0)),
    )(x2d, wq_l, k2d, v2d)


def kernel(x, Wq, K_ext, V_ext, Wo):
    my = lax.axis_index("i")
    sq = x.shape[1]
    hq_per = K_ext.shape[2]
    dh = K_ext.shape[3]
    dcols = hq_per * dh

    Wq_l = lax.dynamic_slice(Wq, (0, my * dcols), (Wq.shape[0], dcols))
    Wo_l = lax.dynamic_slice(Wo, (my * dcols, 0), (dcols, Wo.shape[1]))

    ctx = _sparse_attn(
        x[0],
        Wq_l.astype(jnp.bfloat16),
        K_ext[0].reshape(sq, dcols),
        V_ext[0].reshape(sq, dcols),
        hq_per,
    )

    out = _ring_allreduce(ctx, Wo_l.astype(jnp.bfloat16))
    return out[None]
